# baseline (speedup 1.0000x reference)
import functools

import jax
import jax.numpy as jnp
from jax import lax
from jax.experimental import pallas as pl
from jax.experimental.pallas import tpu as pltpu

_M = 1024
_N = 4
_TOT = _M * _N
_K = 128
_GAMMA = 0.5
_EPS = 0.5
_BSZ = 16
_NCHUNK = 8
_CW = _TOT // _NCHUNK


def _za_body(x_ref, wa_ref, ba_ref, out_ref):
    acc = lax.dot_general(x_ref[...], wa_ref[...],
                          (((1,), (1,)), ((), ())),
                          preferred_element_type=jnp.float32)
    out_ref[...] = acc + ba_ref[...]


def _pred_body(y_ref, wd_ref, bd_ref, out_ref):
    acc = lax.dot_general(y_ref[...], wd_ref[...],
                          (((1,), (1,)), ((), ())),
                          preferred_element_type=jnp.float32)
    out_ref[...] = acc + bd_ref[...]


def _kth_largest_bits(lam_bits, k):
    def body(idx, t):
        b = 30 - idx
        cand = t | (jnp.int32(1) << b)
        cnt = jnp.sum((lam_bits >= cand).astype(jnp.int32))
        return jnp.where(cnt >= k, cand, t)
    return lax.fori_loop(0, 31, body, jnp.int32(0))


def _seq_body(za_ref, wt_ref, bb_ref, ymax_ref, xb_ref, phi_ref, psi_ref,
              xb_s, phi_s, psi_s, z_s):
    i = pl.program_id(0)
    c = pl.program_id(1)

    @pl.when(jnp.logical_and(i == 0, c == 0))
    def _init():
        xb_s[...] = jnp.zeros_like(xb_s)
        phi_s[...] = jnp.zeros_like(phi_s)
        psi_s[...] = jnp.zeros_like(psi_s)

    z_s[:, pl.ds(c * _CW, _CW)] = lax.dot_general(
        xb_s[...], wt_ref[...], (((1,), (0,)), ((), ())),
        preferred_element_type=jnp.float32)

    @pl.when(c == _NCHUNK - 1)
    def _step():
        za = za_ref[0]
        z = z_s[...] + bb_ref[...]
        sig = [z[:, n * _M:(n + 1) * _M] + za for n in range(_N)]
        gmin = jnp.min(jnp.stack([jnp.min(s) for s in sig]))

        phi = [phi_s[:, n * _M:(n + 1) * _M] for n in range(_N)]
        pi = [(1.0 - phi[n]) * (sig[n] - gmin + 1.0) for n in range(_N)]

        best = pi[0]
        nstar = jnp.zeros_like(best, dtype=jnp.int32)
        for n in range(1, _N):
            upd = pi[n] > best
            nstar = jnp.where(upd, jnp.int32(n), nstar)
            best = jnp.maximum(best, pi[n])

        lam_bits = lax.bitcast_convert_type(best, jnp.int32)
        thr = _kth_largest_bits(lam_bits, _K)
        sel = lam_bits >= thr

        ymax = None
        s_tot = jnp.float32(0.0)
        for n in range(_N):
            m = jnp.logical_and(sel, nstar == n).astype(jnp.float32)
            y_n = jnp.tanh(sig[n] * m)
            psi_n = jnp.maximum(psi_s[:, n * _M:(n + 1) * _M] * _EPS, y_n)
            phi_n = jnp.maximum(phi[n] * _GAMMA, y_n)
            psi_s[:, n * _M:(n + 1) * _M] = psi_n
            phi_s[:, n * _M:(n + 1) * _M] = phi_n
            s_tot = s_tot + jnp.sum(psi_n)
            ymax = y_n if ymax is None else jnp.maximum(ymax, y_n)

        ymax_ref[...] = ymax[None]
        alpha = jnp.where(s_tot == 0.0, jnp.float32(1.0), s_tot)
        xb_s[...] = psi_s[...] / alpha

        @pl.when(i == _BSZ - 1)
        def _final():
            xb_ref[...] = xb_s[...]
            phi_ref[...] = phi_s[...]
            psi_ref[...] = psi_s[...]


def _run_seq(za, wt_perm, bb_perm):
    grid = (_BSZ, _NCHUNK)
    out_shapes = (
        jax.ShapeDtypeStruct((_BSZ, 1, _M), jnp.float32),
        jax.ShapeDtypeStruct((1, _TOT), jnp.float32),
        jax.ShapeDtypeStruct((1, _TOT), jnp.float32),
        jax.ShapeDtypeStruct((1, _TOT), jnp.float32),
    )
    return pl.pallas_call(
        _seq_body,
        grid=grid,
        in_specs=[
            pl.BlockSpec((1, 1, _M), lambda i, c: (i, 0, 0)),
            pl.BlockSpec((_TOT, _CW), lambda i, c: (0, c)),
            pl.BlockSpec((1, _TOT), lambda i, c: (0, 0)),
        ],
        out_specs=(
            pl.BlockSpec((1, 1, _M), lambda i, c: (i, 0, 0)),
            pl.BlockSpec((1, _TOT), lambda i, c: (0, 0)),
            pl.BlockSpec((1, _TOT), lambda i, c: (0, 0)),
            pl.BlockSpec((1, _TOT), lambda i, c: (0, 0)),
        ),
        out_shape=out_shapes,
        scratch_shapes=[
            pltpu.VMEM((1, _TOT), jnp.float32),
            pltpu.VMEM((1, _TOT), jnp.float32),
            pltpu.VMEM((1, _TOT), jnp.float32),
            pltpu.VMEM((1, _TOT), jnp.float32),
        ],
        compiler_params=pltpu.CompilerParams(
            dimension_semantics=("arbitrary", "arbitrary")),
    )(za.reshape(_BSZ, 1, _M), wt_perm, bb_perm)


def kernel(batch_x, W_a, b_a, W_b, b_b, W_d, b_d):
    za = pl.pallas_call(
        _za_body,
        out_shape=jax.ShapeDtypeStruct((_BSZ, _M), jnp.float32),
    )(batch_x, W_a, b_a.reshape(1, _M))

    p = jnp.arange(_TOT).reshape(_M, _N).T.reshape(-1)
    wt_perm = W_b.T[p][:, p]
    bb_perm = b_b[p].reshape(1, _TOT)

    ymax, xb_p, phi_p, psi_p = _run_seq(za, wt_perm, bb_perm)

    preds = pl.pallas_call(
        _pred_body,
        out_shape=jax.ShapeDtypeStruct((_BSZ, 1024), jnp.float32),
    )(ymax.reshape(_BSZ, _M), W_d, b_d.reshape(1, 1024))

    xb = xb_p.reshape(_N, _M).T.reshape(-1)
    phi = phi_p.reshape(_N, _M).T
    psi = psi_p.reshape(_N, _M).T
    return preds, xb, phi, psi


# speedup vs baseline: 2.4774x; 2.4774x over previous
import jax
import jax.numpy as jnp
from jax import lax
from jax.experimental import pallas as pl
from jax.experimental.pallas import tpu as pltpu
from jax.experimental.pallas import tpu_sc as plsc

_M = 1024
_N = 4
_TOT = _M * _N
_K = 128
_GAMMA = 0.5
_EPS = 0.5
_BSZ = 16
_NT = 16
_CHUNK = _TOT // _NT
_GPT = _M // _NT
_NV = _CHUNK // 16
_BIG = 3.4e38


def _za_body(x_ref, wa_ref, ba_ref, out_ref):
    acc = lax.dot_general(x_ref[...], wa_ref[...],
                          (((1,), (1,)), ((), ())),
                          preferred_element_type=jnp.float32)
    out_ref[...] = acc + ba_ref[...]


def _pred_body(y_ref, wd_ref, bd_ref, out_ref):
    acc = lax.dot_general(y_ref[...], wd_ref[...],
                          (((1,), (1,)), ((), ())),
                          preferred_element_type=jnp.float32)
    out_ref[...] = acc + bd_ref[...]


def _tanh_via_exp(v):
    return 1.0 - 2.0 / (jnp.exp(2.0 * v) + 1.0)


def _sc_body(za_hbm, wg_hbm, bb_hbm,
             y_hbm, xb_hbm, phi_hbm, psi_hbm,
             za_v, bb_v, u_v, psi_v, phi_v, sig_v,
             sstar_v, lam_v, jstar_v, yrow_v, jj_v, dv_v, dphi_v,
             lamall_v, jall_v, dvall_v,
             cidx_v, cdv_v, rows_v, red_v, tmp_v,
             sh_red, sh_lam, sh_j, sh_dv,
             dma_sem):
    wid = lax.axis_index("s")
    lanes = lax.iota(jnp.int32, 16)
    zeros16 = jnp.zeros((16,), jnp.float32)

    pltpu.sync_copy(bb_hbm.at[wid], bb_v)
    pltpu.sync_copy(za_hbm.at[wid], za_v)
    for k in range(_NV):
        u_v[pl.ds(k * 16, 16)] = zeros16
        psi_v[pl.ds(k * 16, 16)] = zeros16
        phi_v[pl.ds(k * 16, 16)] = zeros16
    for k in range(_K // 16):
        cidx_v[pl.ds(k * 16, 16)] = jnp.zeros((16,), jnp.int32)

    def step(i, s_carry):
        s = s_carry
        alpha = jnp.where(s == 0.0, jnp.float32(1.0), s)
        inv_a = 1.0 / (jnp.full((16,), 1.0) * alpha)

        mnv = jnp.full((16,), _BIG)
        for k in range(_NV):
            g_idx = (k * 16 + lanes) >> 2
            zav = plsc.load_gather(za_v, [i * _GPT + g_idx])
            sg = (zav + u_v[pl.ds(k * 16, 16)] * inv_a
                  + bb_v[pl.ds(k * 16, 16)])
            sig_v[pl.ds(k * 16, 16)] = sg
            mnv = jnp.minimum(mnv, sg)
        tmp_v[...] = mnv
        pltpu.sync_copy(tmp_v, sh_red.at[wid])
        plsc.subcore_barrier()

        pltpu.sync_copy(sh_red, red_v)
        gm = jnp.full((16,), _BIG)
        for t in range(_NT):
            gm = jnp.minimum(gm, red_v[t])
        gmin = jnp.min(gm)

        for k in range(_GPT // 16):
            base = (k * 16 + lanes) * 4
            best = jnp.full((16,), -_BIG)
            beststar = zeros16
            bestj = jnp.zeros((16,), jnp.int32)
            for n in range(_N):
                sgn = plsc.load_gather(sig_v, [base + n])
                phn = plsc.load_gather(phi_v, [base + n])
                pin = (1.0 - phn) * (sgn - gmin + 1.0)
                upd = pin > best
                best = jnp.where(upd, pin, best)
                beststar = jnp.where(upd, sgn, beststar)
                bestj = jnp.where(upd, base + n, bestj)
            lam_v[pl.ds(k * 16, 16)] = best
            sstar_v[pl.ds(k * 16, 16)] = beststar
            jstar_v[pl.ds(k * 16, 16)] = bestj
        pltpu.sync_copy(lam_v, sh_lam.at[pl.ds(wid * _GPT, _GPT)])
        plsc.subcore_barrier()

        pltpu.sync_copy(sh_lam, lamall_v)

        def bit_iter(bi, t_acc):
            cand = t_acc | (jnp.int32(1) << (30 - bi))

            def cnt_iter(k, cv):
                b = plsc.bitcast(lamall_v[pl.ds(k * 16, 16)], jnp.int32)
                return cv + jnp.where(b >= cand, 1.0, 0.0)

            cv = lax.fori_loop(0, _M // 16, cnt_iter, zeros16, unroll=8)
            return jnp.where(jnp.sum(cv) >= jnp.float32(_K), cand, t_acc)

        thr = lax.fori_loop(0, 31, bit_iter, jnp.int32(0))

        for k in range(_GPT // 16):
            sl = pl.ds(k * 16, 16)
            sel = plsc.bitcast(lam_v[sl], jnp.int32) >= thr
            self_f = jnp.where(sel, 1.0, 0.0)
            yv = _tanh_via_exp(sstar_v[sl]) * self_f
            yrow_v[sl] = jnp.maximum(yv, 0.0)
            jst = jstar_v[sl]
            psj = plsc.load_gather(psi_v, [jst])
            phj = plsc.load_gather(phi_v, [jst])
            dv_v[sl] = jnp.maximum(yv - psj * _EPS, 0.0)
            dphi_v[sl] = jnp.maximum(yv - phj * _GAMMA, 0.0)
            jj_v[sl] = jst + wid * _CHUNK
        pltpu.sync_copy(jj_v, sh_j.at[pl.ds(wid * _GPT, _GPT)])
        pltpu.sync_copy(dv_v, sh_dv.at[pl.ds(wid * _GPT, _GPT)])

        for k in range(_NV):
            sl = pl.ds(k * 16, 16)
            psi_v[sl] = psi_v[sl] * _EPS
            phi_v[sl] = phi_v[sl] * _GAMMA
        for k in range(_GPT // 16):
            sl = pl.ds(k * 16, 16)
            jst = jstar_v[sl]
            plsc.addupdate_scatter(psi_v, [jst], dv_v[sl])
            plsc.addupdate_scatter(phi_v, [jst], dphi_v[sl])
        plsc.subcore_barrier()

        pltpu.sync_copy(sh_j, jall_v)
        pltpu.sync_copy(sh_dv, dvall_v)
        for k in range(_K // 16):
            cdv_v[pl.ds(k * 16, 16)] = zeros16

        def comp_iter(k, carry):
            cntf, sumd = carry
            dv = dvall_v[pl.ds(k * 16, 16)]
            jv = jall_v[pl.ds(k * 16, 16)]
            m = dv > 0.0
            mi = jnp.where(m, 1, 0).astype(jnp.int32)
            pos = cntf.astype(jnp.int32) + plsc.cumsum(mi) - 1
            gidx = jv * _NT + wid
            plsc.store_scatter(cidx_v, [pos], gidx, mask=m)
            plsc.store_scatter(cdv_v, [pos], dv, mask=m)
            return (cntf + jnp.sum(jnp.where(m, 1.0, 0.0)),
                    sumd + jnp.sum(dv))

        _, sumd = lax.fori_loop(0, _M // 16, comp_iter,
                                (jnp.float32(0.0), jnp.float32(0.0)))

        pltpu.async_copy(wg_hbm.at[cidx_v], rows_v, dma_sem).wait()

        def acc_iter(r, accs):
            wv = plsc.load_gather(cdv_v, [jnp.full((16,), r, jnp.int32)])
            return tuple(accs[k] + wv * rows_v[r, pl.ds(k * 16, 16)]
                         for k in range(_NV))

        accs = tuple(u_v[pl.ds(k * 16, 16)] * _EPS for k in range(_NV))
        accs = lax.fori_loop(0, _K, acc_iter, accs)
        for k in range(_NV):
            u_v[pl.ds(k * 16, 16)] = accs[k]

        pltpu.sync_copy(yrow_v, y_hbm.at[i, wid])
        return _EPS * s + sumd

    s_fin = lax.fori_loop(0, _BSZ, step, jnp.float32(0.0))

    alpha_f = jnp.where(s_fin == 0.0, jnp.float32(1.0), s_fin)
    inv_f = 1.0 / (jnp.full((16,), 1.0) * alpha_f)
    for k in range(_NV):
        sig_v[pl.ds(k * 16, 16)] = psi_v[pl.ds(k * 16, 16)] * inv_f
    pltpu.sync_copy(sig_v, xb_hbm.at[wid])
    pltpu.sync_copy(phi_v, phi_hbm.at[wid])
    pltpu.sync_copy(psi_v, psi_hbm.at[wid])


def _run_sc(za_t, wg, bb_t):
    mesh = plsc.VectorSubcoreMesh(core_axis_name="c", subcore_axis_name="s",
                                  num_cores=1, num_subcores=_NT)
    f = pl.kernel(
        _sc_body,
        out_type=(
            jax.ShapeDtypeStruct((_BSZ, _NT, _GPT), jnp.float32),
            jax.ShapeDtypeStruct((_NT, _CHUNK), jnp.float32),
            jax.ShapeDtypeStruct((_NT, _CHUNK), jnp.float32),
            jax.ShapeDtypeStruct((_NT, _CHUNK), jnp.float32),
        ),
        mesh=mesh,
        compiler_params=pltpu.CompilerParams(
            needs_layout_passes=False, use_tc_tiling_on_sc=False),
        scratch_types=[
            pltpu.VMEM((_BSZ * _GPT,), jnp.float32),
            pltpu.VMEM((_CHUNK,), jnp.float32),
            pltpu.VMEM((_CHUNK,), jnp.float32),
            pltpu.VMEM((_CHUNK,), jnp.float32),
            pltpu.VMEM((_CHUNK,), jnp.float32),
            pltpu.VMEM((_CHUNK,), jnp.float32),
            pltpu.VMEM((_GPT,), jnp.float32),
            pltpu.VMEM((_GPT,), jnp.float32),
            pltpu.VMEM((_GPT,), jnp.int32),
            pltpu.VMEM((_GPT,), jnp.float32),
            pltpu.VMEM((_GPT,), jnp.int32),
            pltpu.VMEM((_GPT,), jnp.float32),
            pltpu.VMEM((_GPT,), jnp.float32),
            pltpu.VMEM((_M,), jnp.float32),
            pltpu.VMEM((_M,), jnp.int32),
            pltpu.VMEM((_M,), jnp.float32),
            pltpu.VMEM((_K,), jnp.int32),
            pltpu.VMEM((_K,), jnp.float32),
            pltpu.VMEM((_K, _CHUNK), jnp.float32),
            pltpu.VMEM((_NT, 16), jnp.float32),
            pltpu.VMEM((16,), jnp.float32),
            pltpu.VMEM_SHARED((_NT, 16), jnp.float32),
            pltpu.VMEM_SHARED((_M,), jnp.float32),
            pltpu.VMEM_SHARED((_M,), jnp.int32),
            pltpu.VMEM_SHARED((_M,), jnp.float32),
            pltpu.SemaphoreType.DMA,
        ],
    )
    return f(za_t, wg, bb_t)


def kernel(batch_x, W_a, b_a, W_b, b_b, W_d, b_d):
    za = pl.pallas_call(
        _za_body,
        out_shape=jax.ShapeDtypeStruct((_BSZ, _M), jnp.float32),
    )(batch_x, W_a, b_a.reshape(1, _M))

    wg = (W_b.reshape(_NT, _CHUNK, _TOT)
          .transpose(2, 0, 1).reshape(_TOT * _NT, _CHUNK))
    za_t = (za.reshape(_BSZ, _NT, _GPT).transpose(1, 0, 2)
            .reshape(_NT, _BSZ * _GPT))
    bb_t = b_b.reshape(_NT, _CHUNK)

    y_out, xb_out, phi_out, psi_out = _run_sc(za_t, wg, bb_t)

    preds = pl.pallas_call(
        _pred_body,
        out_shape=jax.ShapeDtypeStruct((_BSZ, 1024), jnp.float32),
    )(y_out.reshape(_BSZ, _M), W_d, b_d.reshape(1, 1024))

    xb = xb_out.reshape(_TOT)
    phi = phi_out.reshape(_M, _N)
    psi = psi_out.reshape(_M, _N)
    return preds, xb, phi, psi
